# baseline (speedup 1.0000x reference)
import jax
import jax.numpy as jnp
from jax.experimental import pallas as pl
from jax.experimental.pallas import tpu as pltpu

_CN = 256


def _head_kernel(x_ref, w_ref, b_ref, o_ref):
    nb, fh, fw, t, cin = x_ref.shape
    m = fh * fw * t
    cw = w_ref.shape[0]
    for i in range(nb):
        x2 = x_ref[i].astype(jnp.bfloat16).reshape(m, cin)
        for c in range(0, cw, _CN):
            wc = w_ref[c:c + _CN, :].astype(jnp.bfloat16)
            y = jax.lax.dot_general(
                x2, wc, (((1,), (1,)), ((), ())),
                preferred_element_type=jnp.float32)
            y = y + b_ref[:, c:c + _CN]
            o_ref[i, :, :, :, c:c + _CN] = y.reshape(fh, fw, t, _CN)


def kernel(features, weight, bias):
    B, T, Cin, fh, fw = features.shape
    Cout = weight.shape[0]

    xp = jnp.transpose(features, (0, 3, 4, 1, 2))
    b2d = bias.reshape(1, Cout)

    nb = 2 if B % 2 == 0 else 1
    csp = 2 if Cout % 512 == 0 else 1
    cb = Cout // csp
    out = pl.pallas_call(
        _head_kernel,
        out_shape=jax.ShapeDtypeStruct((B, fh, fw, T, Cout), features.dtype),
        grid=(B // nb, csp),
        in_specs=[
            pl.BlockSpec((nb, fh, fw, T, Cin), lambda b, s: (b, 0, 0, 0, 0)),
            pl.BlockSpec((cb, Cin), lambda b, s: (s, 0)),
            pl.BlockSpec((1, cb), lambda b, s: (0, s)),
        ],
        out_specs=pl.BlockSpec((nb, fh, fw, T, cb), lambda b, s: (b, 0, 0, 0, s)),
        compiler_params=pltpu.CompilerParams(
            dimension_semantics=("parallel", "arbitrary")),
        name="conv1x1_head",
    )(xp, weight, b2d)
    return jnp.transpose(out, (0, 3, 4, 1, 2))


# speedup vs baseline: 1.3621x; 1.3621x over previous
import jax
import jax.numpy as jnp
from jax.experimental import pallas as pl
from jax.experimental.pallas import tpu as pltpu

_CN = 256


def _head_kernel(x_ref, w_ref, b_ref, o_ref):
    nb, fh, fw, t, cin = x_ref.shape
    m = fh * fw * t
    cout = w_ref.shape[0]
    for i in range(nb):
        x2 = x_ref[i].astype(jnp.bfloat16).reshape(m, cin)
        for c in range(0, cout, _CN):
            wc = w_ref[c:c + _CN, :].astype(jnp.bfloat16)
            y = jax.lax.dot_general(
                x2, wc, (((1,), (1,)), ((), ())),
                preferred_element_type=jnp.float32)
            y = y + b_ref[:, c:c + _CN]
            o_ref[i, :, :, :, c:c + _CN] = y.reshape(fh, fw, t, _CN)


def kernel(features, weight, bias):
    B, T, Cin, fh, fw = features.shape
    Cout = weight.shape[0]

    xp = jnp.transpose(features, (0, 3, 4, 1, 2))
    b2d = bias.reshape(1, Cout)

    nb = 2 if B % 2 == 0 else 1
    out = pl.pallas_call(
        _head_kernel,
        out_shape=jax.ShapeDtypeStruct((B, fh, fw, T, Cout), features.dtype),
        grid=(B // nb,),
        in_specs=[
            pl.BlockSpec((nb, fh, fw, T, Cin), lambda b: (b, 0, 0, 0, 0)),
            pl.BlockSpec((Cout, Cin), lambda b: (0, 0)),
            pl.BlockSpec((1, Cout), lambda b: (0, 0)),
        ],
        out_specs=pl.BlockSpec((nb, fh, fw, T, Cout), lambda b: (b, 0, 0, 0, 0)),
        compiler_params=pltpu.CompilerParams(
            dimension_semantics=("parallel",)),
        name="conv1x1_head",
    )(xp, weight, b2d)
    return jnp.transpose(out, (0, 3, 4, 1, 2))


# speedup vs baseline: 1.3677x; 1.0042x over previous
import jax
import jax.numpy as jnp
from jax.experimental import pallas as pl
from jax.experimental.pallas import tpu as pltpu

_CN = 256


def _head_kernel(x_ref, w_ref, b_ref, o_ref):
    nb, fh, fw, t, cin = x_ref.shape
    m = fh * fw * t
    cout = w_ref.shape[0]
    for i in range(nb):
        x2 = x_ref[i].astype(jnp.bfloat16).reshape(m, cin)
        for c in range(0, cout, _CN):
            cn = min(_CN, cout - c)
            wc = w_ref[c:c + cn, :].astype(jnp.bfloat16)
            y = jax.lax.dot_general(
                x2, wc, (((1,), (1,)), ((), ())),
                preferred_element_type=jnp.float32)
            y = y + b_ref[:, c:c + cn]
            o_ref[i, :, :, :, c:c + cn] = y.reshape(fh, fw, t, cn)


def kernel(features, weight, bias):
    B, T, Cin, fh, fw = features.shape
    Cout = weight.shape[0]

    xp = jnp.transpose(features, (0, 3, 4, 1, 2))
    b2d = bias.reshape(1, Cout)

    nb = 2 if B % 2 == 0 else 1
    out = pl.pallas_call(
        _head_kernel,
        out_shape=jax.ShapeDtypeStruct((B, fh, fw, T, Cout), features.dtype),
        grid=(B // nb,),
        in_specs=[
            pl.BlockSpec((nb, fh, fw, T, Cin), lambda b: (b, 0, 0, 0, 0)),
            pl.BlockSpec((Cout, Cin), lambda b: (0, 0)),
            pl.BlockSpec((1, Cout), lambda b: (0, 0)),
        ],
        out_specs=pl.BlockSpec((nb, fh, fw, T, Cout), lambda b: (b, 0, 0, 0, 0)),
        compiler_params=pltpu.CompilerParams(
            dimension_semantics=("parallel",)),
        name="conv1x1_head",
    )(xp, weight, b2d)
    return jnp.transpose(out, (0, 3, 4, 1, 2))
